# baseline (speedup 1.0000x reference)
import functools

import jax
import jax.numpy as jnp
from jax import lax
from jax.experimental import pallas as pl
from jax.experimental.pallas import tpu as pltpu
from jax.experimental.pallas import tpu_sc as plsc

N = 10000
E = 320000
D = 128

NC = 2
NS = 16
NW = NC * NS
CHUNK = 128
CPT = 80
BLK = 8
E_PAD = NW * CPT * CHUNK
N_PAD = 10112
ROWS_PT = N_PAD // NS
LANES = 16


_ROWS_BLK = 2000


def _pre_body(x_ref, wn_ref, ws_ref, b_ref, y_ref, h_ref):
    x = x_ref[...]
    y_ref[...] = lax.dot_general(
        x, wn_ref[...], (((1,), (1,)), ((), ())),
        preferred_element_type=jnp.float32)
    h_ref[...] = lax.dot_general(
        x, ws_ref[...], (((1,), (1,)), ((), ())),
        preferred_element_type=jnp.float32) + b_ref[...]


@jax.jit
def _pre(x, W_neigh, W_self, b2d):
    return pl.pallas_call(
        _pre_body,
        grid=(N // _ROWS_BLK,),
        in_specs=[
            pl.BlockSpec((_ROWS_BLK, D), lambda i: (i, 0)),
            pl.BlockSpec((D, D), lambda i: (0, 0)),
            pl.BlockSpec((D, D), lambda i: (0, 0)),
            pl.BlockSpec((1, D), lambda i: (0, 0)),
        ],
        out_specs=[
            pl.BlockSpec((_ROWS_BLK, D), lambda i: (i, 0)),
            pl.BlockSpec((_ROWS_BLK, D), lambda i: (i, 0)),
        ],
        out_shape=[
            jax.ShapeDtypeStruct((N, D), jnp.float32),
            jax.ShapeDtypeStruct((N, D), jnp.float32),
        ],
    )(x, W_neigh, W_self, b2d)


_sc_mesh = plsc.VectorSubcoreMesh(core_axis_name="c", subcore_axis_name="s")


@functools.partial(
    pl.kernel,
    out_type=(
        jax.ShapeDtypeStruct((NC, N_PAD, D), jnp.float32),
        jax.ShapeDtypeStruct((NC, N_PAD), jnp.float32),
    ),
    mesh=_sc_mesh,
    scratch_types=[
        pltpu.VMEM((BLK, CHUNK), jnp.int32),
        pltpu.VMEM((BLK, CHUNK), jnp.int32),
        pltpu.VMEM((CHUNK, D), jnp.float32),
        pltpu.VMEM((CHUNK, D), jnp.float32),
        pltpu.VMEM((CHUNK,), jnp.float32),
        pltpu.VMEM_SHARED((N_PAD, D), jnp.float32),
        pltpu.VMEM_SHARED((N_PAD,), jnp.float32),
        pltpu.SemaphoreType.DMA,
        pltpu.SemaphoreType.DMA,
    ],
)
def _sc_seg(y_hbm, src_hbm, dst_hbm, z2_hbm, z1_hbm, seg_out, deg_out,
            src_blk, dst_blk, buf0, buf1, ones_v, acc_sh, deg_sh, sg0, sg1):
    c = lax.axis_index("c")
    s = lax.axis_index("s")
    w = s * NC + c

    pltpu.sync_copy(z2_hbm.at[pl.ds(s * ROWS_PT, ROWS_PT)],
                    acc_sh.at[pl.ds(s * ROWS_PT, ROWS_PT)])

    @pl.when(s == 0)
    def _():
        pltpu.sync_copy(z1_hbm, deg_sh)

    for i in range(CHUNK // LANES):
        ones_v[pl.ds(i * LANES, LANES)] = jnp.ones((LANES,), jnp.float32)

    plsc.subcore_barrier()

    bufs = (buf0, buf1)
    sems = (sg0, sg1)

    def body(b, carry):
        row = (w * CPT + b * BLK)
        pltpu.sync_copy(src_hbm.at[pl.ds(row, BLK)], src_blk)
        pltpu.sync_copy(dst_hbm.at[pl.ds(row, BLK)], dst_blk)
        for k in range(BLK):
            pltpu.async_copy(y_hbm.at[src_blk.at[k]], bufs[0], sems[0]).wait()
            pltpu.sync_copy(bufs[0], acc_sh.at[dst_blk.at[k]], add=True)
            pltpu.sync_copy(ones_v, deg_sh.at[dst_blk.at[k]], add=True)
        return carry

    lax.fori_loop(0, CPT // BLK, body, 0)

    plsc.subcore_barrier()

    pltpu.sync_copy(acc_sh.at[pl.ds(s * ROWS_PT, ROWS_PT)],
                    seg_out.at[c, pl.ds(s * ROWS_PT, ROWS_PT)])

    @pl.when(s == 0)
    def _():
        pltpu.sync_copy(deg_sh, deg_out.at[c])


def _post_body(h_ref, seg_ref, deg_ref, o_ref):
    ssum = seg_ref[0] + seg_ref[1]
    dsum = deg_ref[0] + deg_ref[1]
    o_ref[...] = h_ref[...] + ssum / jnp.maximum(dsum, 1.0)


@jax.jit
def _post(h_self, seg, deg3):
    return pl.pallas_call(
        _post_body,
        grid=(N // _ROWS_BLK,),
        in_specs=[
            pl.BlockSpec((_ROWS_BLK, D), lambda i: (i, 0)),
            pl.BlockSpec((NC, _ROWS_BLK, D), lambda i: (0, i, 0)),
            pl.BlockSpec((NC, _ROWS_BLK, 1), lambda i: (0, i, 0)),
        ],
        out_specs=pl.BlockSpec((_ROWS_BLK, D), lambda i: (i, 0)),
        out_shape=jax.ShapeDtypeStruct((N, D), jnp.float32),
    )(h_self, seg, deg3)


def kernel(x, edge_index, W_neigh, W_self, b_self):
    pad = E_PAD - E
    src = jnp.concatenate(
        [edge_index[0], jnp.zeros((pad,), jnp.int32)]).reshape(-1, CHUNK)
    dst = jnp.concatenate(
        [edge_index[1], jnp.full((pad,), N, jnp.int32)]).reshape(-1, CHUNK)
    y, h_self = _pre(x, W_neigh, W_self, b_self.reshape(1, D))
    z2 = jnp.zeros((N_PAD, D), jnp.float32)
    z1 = jnp.zeros((N_PAD,), jnp.float32)
    seg, deg = _sc_seg(y, src, dst, z2, z1)
    return _post(h_self, seg, deg.reshape(NC, N_PAD, 1))


# speedup vs baseline: 1.1023x; 1.1023x over previous
import functools

import jax
import jax.numpy as jnp
from jax import lax
from jax.experimental import pallas as pl
from jax.experimental.pallas import tpu as pltpu
from jax.experimental.pallas import tpu_sc as plsc

N = 10000
E = 320000
D = 128

NC = 2
NS = 16
NW = NC * NS
CHUNK = 128
CPT = 80
BLK = 8
E_PAD = NW * CPT * CHUNK
N_PAD = 10112
ROWS_PT = N_PAD // NS
LANES = 16


_ROWS_BLK = 2000


def _pre_body(x_ref, wn_ref, ws_ref, b_ref, y_ref, h_ref):
    x = x_ref[...]
    y_ref[...] = lax.dot_general(
        x, wn_ref[...], (((1,), (1,)), ((), ())),
        preferred_element_type=jnp.float32)
    h_ref[...] = lax.dot_general(
        x, ws_ref[...], (((1,), (1,)), ((), ())),
        preferred_element_type=jnp.float32) + b_ref[...]


@jax.jit
def _pre(x, W_neigh, W_self, b2d):
    return pl.pallas_call(
        _pre_body,
        grid=(N // _ROWS_BLK,),
        in_specs=[
            pl.BlockSpec((_ROWS_BLK, D), lambda i: (i, 0)),
            pl.BlockSpec((D, D), lambda i: (0, 0)),
            pl.BlockSpec((D, D), lambda i: (0, 0)),
            pl.BlockSpec((1, D), lambda i: (0, 0)),
        ],
        out_specs=[
            pl.BlockSpec((_ROWS_BLK, D), lambda i: (i, 0)),
            pl.BlockSpec((_ROWS_BLK, D), lambda i: (i, 0)),
        ],
        out_shape=[
            jax.ShapeDtypeStruct((N, D), jnp.float32),
            jax.ShapeDtypeStruct((N, D), jnp.float32),
        ],
    )(x, W_neigh, W_self, b2d)


_sc_mesh = plsc.VectorSubcoreMesh(core_axis_name="c", subcore_axis_name="s")


@functools.partial(
    pl.kernel,
    out_type=(
        jax.ShapeDtypeStruct((NC, N_PAD, D), jnp.float32),
        jax.ShapeDtypeStruct((NC, N_PAD), jnp.float32),
    ),
    mesh=_sc_mesh,
    scratch_types=[
        pltpu.VMEM((CHUNK,), jnp.int32),
        pltpu.VMEM((CHUNK,), jnp.int32),
        pltpu.VMEM((CHUNK,), jnp.int32),
        pltpu.VMEM((CHUNK,), jnp.int32),
        pltpu.VMEM((CHUNK, D), jnp.float32),
        pltpu.VMEM((CHUNK, D), jnp.float32),
        pltpu.VMEM((CHUNK,), jnp.float32),
        pltpu.VMEM_SHARED((N_PAD, D), jnp.float32),
        pltpu.VMEM_SHARED((N_PAD,), jnp.float32),
        pltpu.SemaphoreType.DMA,
        pltpu.SemaphoreType.DMA,
    ],
)
def _sc_seg(y_hbm, src_hbm, dst_hbm, z2_hbm, z1_hbm, seg_out, deg_out,
            srcA, dstA, srcB, dstB, bufA, bufB, ones_v, acc_sh, deg_sh,
            sgA, sgB):
    c = lax.axis_index("c")
    s = lax.axis_index("s")
    w = s * NC + c

    pltpu.sync_copy(z2_hbm.at[pl.ds(s * ROWS_PT, ROWS_PT)],
                    acc_sh.at[pl.ds(s * ROWS_PT, ROWS_PT)])

    @pl.when(s == 0)
    def _():
        pltpu.sync_copy(z1_hbm, deg_sh)

    for i in range(CHUNK // LANES):
        ones_v[pl.ds(i * LANES, LANES)] = jnp.ones((LANES,), jnp.float32)

    plsc.subcore_barrier()

    base = w * CPT * CHUNK

    pltpu.sync_copy(src_hbm.at[pl.ds(base, CHUNK)], srcA)
    pltpu.sync_copy(dst_hbm.at[pl.ds(base, CHUNK)], dstA)
    pltpu.async_copy(y_hbm.at[srcA], bufA, sgA)

    def body(i, carry):
        e0 = base + 2 * i * CHUNK
        pltpu.sync_copy(src_hbm.at[pl.ds(e0 + CHUNK, CHUNK)], srcB)
        pltpu.sync_copy(dst_hbm.at[pl.ds(e0 + CHUNK, CHUNK)], dstB)
        pltpu.async_copy(y_hbm.at[srcB], bufB, sgB)
        pltpu.make_async_copy(y_hbm.at[srcA], bufA, sgA).wait()
        pltpu.sync_copy(bufA, acc_sh.at[dstA], add=True)
        pltpu.sync_copy(ones_v, deg_sh.at[dstA], add=True)

        @pl.when(i + 1 < CPT // 2)
        def _():
            pltpu.sync_copy(src_hbm.at[pl.ds(e0 + 2 * CHUNK, CHUNK)], srcA)
            pltpu.sync_copy(dst_hbm.at[pl.ds(e0 + 2 * CHUNK, CHUNK)], dstA)
            pltpu.async_copy(y_hbm.at[srcA], bufA, sgA)

        pltpu.make_async_copy(y_hbm.at[srcB], bufB, sgB).wait()
        pltpu.sync_copy(bufB, acc_sh.at[dstB], add=True)
        pltpu.sync_copy(ones_v, deg_sh.at[dstB], add=True)
        return carry

    lax.fori_loop(0, CPT // 2, body, 0)

    plsc.subcore_barrier()

    pltpu.sync_copy(acc_sh.at[pl.ds(s * ROWS_PT, ROWS_PT)],
                    seg_out.at[c, pl.ds(s * ROWS_PT, ROWS_PT)])

    @pl.when(s == 0)
    def _():
        pltpu.sync_copy(deg_sh, deg_out.at[c])


def _post_body(h_ref, seg_ref, deg_ref, o_ref):
    ssum = seg_ref[0] + seg_ref[1]
    dsum = deg_ref[0] + deg_ref[1]
    o_ref[...] = h_ref[...] + ssum / jnp.maximum(dsum, 1.0)


@jax.jit
def _post(h_self, seg, deg3):
    return pl.pallas_call(
        _post_body,
        grid=(N // _ROWS_BLK,),
        in_specs=[
            pl.BlockSpec((_ROWS_BLK, D), lambda i: (i, 0)),
            pl.BlockSpec((NC, _ROWS_BLK, D), lambda i: (0, i, 0)),
            pl.BlockSpec((NC, _ROWS_BLK, 1), lambda i: (0, i, 0)),
        ],
        out_specs=pl.BlockSpec((_ROWS_BLK, D), lambda i: (i, 0)),
        out_shape=jax.ShapeDtypeStruct((N, D), jnp.float32),
    )(h_self, seg, deg3)


def kernel(x, edge_index, W_neigh, W_self, b_self):
    pad = E_PAD - E
    src = jnp.concatenate([edge_index[0], jnp.zeros((pad,), jnp.int32)])
    dst = jnp.concatenate([edge_index[1], jnp.full((pad,), N, jnp.int32)])
    y, h_self = _pre(x, W_neigh, W_self, b_self.reshape(1, D))
    z2 = jnp.zeros((N_PAD, D), jnp.float32)
    z1 = jnp.zeros((N_PAD,), jnp.float32)
    seg, deg = _sc_seg(y, src, dst, z2, z1)
    return _post(h_self, seg, deg.reshape(NC, N_PAD, 1))


# speedup vs baseline: 1.1230x; 1.0187x over previous
import functools

import jax
import jax.numpy as jnp
from jax import lax
from jax.experimental import pallas as pl
from jax.experimental.pallas import tpu as pltpu
from jax.experimental.pallas import tpu_sc as plsc

N = 10000
E = 320000
D = 128

NC = 2
NS = 16
NW = NC * NS
CHUNK = 128
CPT = 80
BLK = 8
E_PAD = NW * CPT * CHUNK
N_PAD = 10112
ROWS_PT = N_PAD // NS
LANES = 16


_ROWS_BLK = 2000


def _pre_body(x_ref, wn_ref, ws_ref, b_ref, y_ref, h_ref):
    x = x_ref[...]
    y_ref[...] = lax.dot_general(
        x, wn_ref[...], (((1,), (1,)), ((), ())),
        preferred_element_type=jnp.float32)
    h_ref[...] = lax.dot_general(
        x, ws_ref[...], (((1,), (1,)), ((), ())),
        preferred_element_type=jnp.float32) + b_ref[...]


@jax.jit
def _pre(x, W_neigh, W_self, b2d):
    return pl.pallas_call(
        _pre_body,
        grid=(N // _ROWS_BLK,),
        in_specs=[
            pl.BlockSpec((_ROWS_BLK, D), lambda i: (i, 0)),
            pl.BlockSpec((D, D), lambda i: (0, 0)),
            pl.BlockSpec((D, D), lambda i: (0, 0)),
            pl.BlockSpec((1, D), lambda i: (0, 0)),
        ],
        out_specs=[
            pl.BlockSpec((_ROWS_BLK, D), lambda i: (i, 0)),
            pl.BlockSpec((_ROWS_BLK, D), lambda i: (i, 0)),
        ],
        out_shape=[
            jax.ShapeDtypeStruct((N, D), jnp.float32),
            jax.ShapeDtypeStruct((N, D), jnp.float32),
        ],
    )(x, W_neigh, W_self, b2d)


_sc_mesh = plsc.VectorSubcoreMesh(core_axis_name="c", subcore_axis_name="s")


@functools.partial(
    pl.kernel,
    out_type=(
        jax.ShapeDtypeStruct((NC, N_PAD, D), jnp.float32),
        jax.ShapeDtypeStruct((NC, N_PAD), jnp.float32),
    ),
    mesh=_sc_mesh,
    scratch_types=[
        pltpu.VMEM((CHUNK,), jnp.int32),
        pltpu.VMEM((CHUNK,), jnp.int32),
        pltpu.VMEM((CHUNK,), jnp.int32),
        pltpu.VMEM((CHUNK,), jnp.int32),
        pltpu.VMEM((CHUNK, D), jnp.float32),
        pltpu.VMEM((CHUNK, D), jnp.float32),
        pltpu.VMEM((CHUNK,), jnp.float32),
        pltpu.VMEM_SHARED((N_PAD, D), jnp.float32),
        pltpu.VMEM_SHARED((N_PAD,), jnp.float32),
        pltpu.SemaphoreType.DMA,
        pltpu.SemaphoreType.DMA,
    ],
)
def _sc_seg(y_hbm, src_hbm, dst_hbm, z2_hbm, z1_hbm, seg_out, deg_out,
            srcA, dstA, srcB, dstB, bufA, bufB, ones_v, acc_sh, deg_sh,
            sgA, sgB):
    c = lax.axis_index("c")
    s = lax.axis_index("s")
    w = s * NC + c

    pltpu.sync_copy(z2_hbm.at[pl.ds(s * ROWS_PT, ROWS_PT)],
                    acc_sh.at[pl.ds(s * ROWS_PT, ROWS_PT)])

    @pl.when(s == 0)
    def _():
        pltpu.sync_copy(z1_hbm, deg_sh)

    for i in range(CHUNK // LANES):
        ones_v[pl.ds(i * LANES, LANES)] = jnp.ones((LANES,), jnp.float32)

    plsc.subcore_barrier()

    base = w * CPT * CHUNK

    pltpu.sync_copy(src_hbm.at[pl.ds(base, CHUNK)], srcA)
    pltpu.sync_copy(dst_hbm.at[pl.ds(base, CHUNK)], dstA)
    pltpu.async_copy(y_hbm.at[srcA], bufA, sgA)

    def body(i, carry):
        e0 = base + 2 * i * CHUNK
        pltpu.sync_copy(src_hbm.at[pl.ds(e0 + CHUNK, CHUNK)], srcB)
        pltpu.sync_copy(dst_hbm.at[pl.ds(e0 + CHUNK, CHUNK)], dstB)
        pltpu.async_copy(y_hbm.at[srcB], bufB, sgB)
        pltpu.make_async_copy(y_hbm.at[srcA], bufA, sgA).wait()
        pltpu.sync_copy(bufA, acc_sh.at[dstA], add=True)
        pltpu.sync_copy(ones_v, deg_sh.at[dstA], add=True)

        @pl.when(i + 1 < CPT // 2)
        def _():
            pltpu.sync_copy(src_hbm.at[pl.ds(e0 + 2 * CHUNK, CHUNK)], srcA)
            pltpu.sync_copy(dst_hbm.at[pl.ds(e0 + 2 * CHUNK, CHUNK)], dstA)
            pltpu.async_copy(y_hbm.at[srcA], bufA, sgA)

        pltpu.make_async_copy(y_hbm.at[srcB], bufB, sgB).wait()
        pltpu.sync_copy(bufB, acc_sh.at[dstB], add=True)
        pltpu.sync_copy(ones_v, deg_sh.at[dstB], add=True)
        return carry

    lax.fori_loop(0, CPT // 2, body, 0)

    plsc.subcore_barrier()

    pltpu.sync_copy(acc_sh.at[pl.ds(s * ROWS_PT, ROWS_PT)],
                    seg_out.at[c, pl.ds(s * ROWS_PT, ROWS_PT)])

    @pl.when(s == 0)
    def _():
        pltpu.sync_copy(deg_sh, deg_out.at[c])


def _post_body(h_ref, seg_ref, deg_ref, o_ref):
    ssum = seg_ref[0] + seg_ref[1]
    dsum = deg_ref[0] + deg_ref[1]
    o_ref[...] = h_ref[...] + ssum / jnp.maximum(dsum, 1.0)


@jax.jit
def _post(h_self, seg, deg3):
    return pl.pallas_call(
        _post_body,
        grid=(N // _ROWS_BLK,),
        in_specs=[
            pl.BlockSpec((_ROWS_BLK, D), lambda i: (i, 0)),
            pl.BlockSpec((NC, _ROWS_BLK, D), lambda i: (0, i, 0)),
            pl.BlockSpec((NC, _ROWS_BLK, 1), lambda i: (0, i, 0)),
        ],
        out_specs=pl.BlockSpec((_ROWS_BLK, D), lambda i: (i, 0)),
        out_shape=jax.ShapeDtypeStruct((N, D), jnp.float32),
    )(h_self, seg, deg3)


def kernel(x, edge_index, W_neigh, W_self, b_self):
    pad = E_PAD - E
    trash = N + jnp.arange(pad, dtype=jnp.int32) % (N_PAD - N)
    src = jnp.concatenate([edge_index[0], jnp.zeros((pad,), jnp.int32)])
    dst = jnp.concatenate([edge_index[1], trash])
    y, h_self = _pre(x, W_neigh, W_self, b_self.reshape(1, D))
    z2 = jnp.zeros((N_PAD, D), jnp.float32)
    z1 = jnp.zeros((N_PAD,), jnp.float32)
    seg, deg = _sc_seg(y, src, dst, z2, z1)
    return _post(h_self, seg, deg.reshape(NC, N_PAD, 1))
